# baseline (speedup 1.0000x reference)
import jax
import jax.numpy as jnp
from jax.experimental import pallas as pl
from jax.experimental.pallas import tpu as pltpu

DIM = 1024
HID = 1024
E = 16
T = 16
LIMIT = 7.0
EPS = 1e-5

C1 = 2
C2 = 1
R1 = HID // C1
R2 = DIM // C2


def _router(x_ref, nw_ref, gw_ref, gb_ref, xn_ref, wd_ref, sched_ref):
    x = x_ref[...]
    ms = jnp.mean(x * x, axis=1, keepdims=True)
    xn = x * jax.lax.rsqrt(ms + EPS) * nw_ref[...]
    xn_ref[...] = xn
    g = jax.lax.dot_general(xn, gw_ref[...], (((1,), (1,)), ((), ())),
                            preferred_element_type=jnp.float32)
    g = g + gb_ref[...]
    iota = jax.lax.broadcasted_iota(jnp.int32, (T, E), 1)
    m1 = jnp.max(g, axis=1, keepdims=True)
    idx1 = jnp.min(jnp.where(g == m1, iota, E), axis=1, keepdims=True)
    g2 = jnp.where(iota == idx1, -jnp.inf, g)
    m2 = jnp.max(g2, axis=1, keepdims=True)
    idx2 = jnp.min(jnp.where(g2 == m2, iota, E), axis=1, keepdims=True)
    e2 = jnp.exp(m2 - m1)
    denom = 1.0 + e2
    wd = (jnp.where(iota == idx1, 1.0 / denom, 0.0)
          + jnp.where(iota == idx2, e2 / denom, 0.0))
    wd_ref[...] = wd

    ones_col = jnp.ones((T, 1), jnp.float32)
    sums_col = jax.lax.dot_general(wd, ones_col, (((0,), (0,)), ((), ())),
                                   preferred_element_type=jnp.float32)
    used_col = (sums_col > 0.0).astype(jnp.float32)
    r_iota = jax.lax.broadcasted_iota(jnp.int32, (E, E), 0)
    c_iota = jax.lax.broadcasted_iota(jnp.int32, (E, E), 1)
    tril = (c_iota <= r_iota).astype(jnp.float32)
    pos_col = jax.lax.dot_general(tril, used_col, (((1,), (0,)), ((), ())),
                                  preferred_element_type=jnp.float32)
    posm1 = pos_col - 1.0
    assign = jnp.where((posm1 == c_iota.astype(jnp.float32))
                       & (used_col > 0.0), 1.0, 0.0)
    e_row = jax.lax.broadcasted_iota(jnp.int32, (1, E), 1).astype(jnp.float32)
    sched_row = jax.lax.dot_general(e_row, assign, (((1,), (0,)), ((), ())),
                                    preferred_element_type=jnp.float32)
    n = jnp.sum(used_col)
    e_col = jax.lax.broadcasted_iota(jnp.int32, (E, 1), 0).astype(jnp.float32)
    last = jnp.max(jnp.where(used_col > 0.0, e_col, -1.0))
    sched_full = jnp.where(e_row < n, sched_row, last)
    out_rows = jax.lax.broadcasted_iota(jnp.int32, (8, E), 0)
    sched_ref[...] = jnp.where(out_rows == 0,
                               jnp.broadcast_to(sched_full, (8, E)),
                               n).astype(jnp.int32)


def _sweep(sched_ref, x_ref, xn_ref, wd_ref, *rest):
    w1_refs = rest[:C1]
    w2_refs = rest[C1:C1 + C2]
    b1g_ref, b1l_ref, b2_ref, out_ref = rest[C1 + C2:]
    i = pl.program_id(0)
    n = sched_ref[1, 0]
    e = sched_ref[0, i]

    @pl.when(i == 0)
    def _init():
        out_ref[...] = x_ref[...]

    @pl.when(i < n)
    def _accum():
        xn = xn_ref[...]
        acts = []
        for c in range(C1):
            w1c = w1_refs[c][0]
            hg = jax.lax.dot_general(xn, w1c[:, :DIM],
                                     (((1,), (1,)), ((), ())),
                                     preferred_element_type=jnp.float32)
            hg = hg + b1g_ref[0][:, c * R1:(c + 1) * R1]
            hl = jax.lax.dot_general(xn, w1c[:, DIM:],
                                     (((1,), (1,)), ((), ())),
                                     preferred_element_type=jnp.float32)
            hl = hl + b1l_ref[0][:, c * R1:(c + 1) * R1]
            hg = jnp.minimum(hg, LIMIT)
            hl = jnp.clip(hl, -LIMIT, LIMIT)
            acts.append(hg * jax.nn.sigmoid(1.702 * hg) * (hl + 1.0))
        act = jnp.concatenate(acts, axis=1) if C1 > 1 else acts[0]
        ys = [jax.lax.dot_general(act, w2_refs[c][0],
                                  (((1,), (1,)), ((), ())),
                                  preferred_element_type=jnp.float32)
              for c in range(C2)]
        y = (jnp.concatenate(ys, axis=1) if C2 > 1 else ys[0]) + b2_ref[0]
        iota = jax.lax.broadcasted_iota(jnp.int32, (T, E), 1)
        wcol = jnp.sum(jnp.where(iota == e, wd_ref[...], 0.0), axis=1,
                       keepdims=True)
        out_ref[...] += wcol * y


def kernel(x, freqs_cos, freqs_sin, gate_w, gate_b, w1, b1, w2, b2, norm_w):
    del freqs_cos, freqs_sin
    w1r = w1.reshape(E, HID, 2 * DIM)
    b1g = b1[:, 0::2].reshape(E, 1, HID)
    b1l = b1[:, 1::2].reshape(E, 1, HID)
    b2r = b2.reshape(E, 1, DIM)
    nw = norm_w.reshape(1, DIM)
    gb = gate_b.reshape(1, E)

    xn, wd, sched = pl.pallas_call(
        _router,
        out_shape=[
            jax.ShapeDtypeStruct((T, DIM), jnp.float32),
            jax.ShapeDtypeStruct((T, E), jnp.float32),
            jax.ShapeDtypeStruct((8, E), jnp.int32),
        ],
    )(x, nw, gate_w, gb)

    full = lambda shape: pl.BlockSpec(shape, lambda i, s: (0,) * len(shape))
    per_e2 = lambda s1: pl.BlockSpec((1,) + s1, lambda i, s: (s[0, i], 0, 0))
    w1_specs = [pl.BlockSpec((1, R1, 2 * DIM),
                             lambda i, s, c=c: (s[0, i], c, 0))
                for c in range(C1)]
    w2_specs = [pl.BlockSpec((1, R2, HID),
                             lambda i, s, c=c: (s[0, i], c, 0))
                for c in range(C2)]

    grid_spec = pltpu.PrefetchScalarGridSpec(
        num_scalar_prefetch=1,
        grid=(E,),
        in_specs=(
            [full((T, DIM)),
             full((T, DIM)),
             full((T, E))]
            + w1_specs + w2_specs +
            [per_e2((1, HID)),
             per_e2((1, HID)),
             per_e2((1, DIM))]
        ),
        out_specs=full((T, DIM)),
    )

    return pl.pallas_call(
        _sweep,
        grid_spec=grid_spec,
        out_shape=jax.ShapeDtypeStruct((T, DIM), jnp.float32),
        compiler_params=pltpu.CompilerParams(
            dimension_semantics=("arbitrary",),
        ),
    )(sched, x, xn, wd, *([w1r] * C1), *([w2] * C2), b1g, b1l, b2r)
